# baseline (speedup 1.0000x reference)
import functools

import jax
import jax.numpy as jnp
from jax import lax
from jax.experimental import pallas as pl
from jax.experimental.pallas import tpu as pltpu
from jax.experimental.pallas import tpu_sc as plsc

_HEADS = 4
_HID = 128
_D = 128
_N = 10000
_E = 320000
_NPAD = 10240
_EPAD = 327680
_CHUNK = 128
_NCH = _EPAD // _CHUNK
_NC = 2
_NS = 16
_NW = _NC * _NS
_CPW = _NCH // _NW
_CPT = _NCH // _NC // _NS
_RB = _NPAD // _NS
_DCH = 32
_ACH = 80
_GRID = _NPAD // 640


def _sc_mesh():
    return plsc.VectorSubcoreMesh(
        core_axis_name="c", subcore_axis_name="s",
        num_cores=_NC, num_subcores=_NS)

_SC_PARAMS = dict(
    compiler_params=pltpu.CompilerParams(needs_layout_passes=False))


def _fill_rows(ref, rows, width, value):
    def body(i, _):
        for q in range(width // 16):
            ref[i, pl.ds(q * 16, 16)] = jnp.full((16,), value, jnp.float32)
        return 0
    lax.fori_loop(0, rows, body, 0)


def _zero_acc_slice(zrows_v, acc_sp, s):
    for r in range(_RB // _CHUNK):
        pltpu.sync_copy(zrows_v,
                        acc_sp.at[pl.ds(s * _RB + r * _CHUNK, _CHUNK)])


def _sc_hist(dstp):
    @functools.partial(
        pl.kernel,
        out_type=jax.ShapeDtypeStruct((_NC * _NPAD, _D), jnp.float32),
        mesh=_sc_mesh(),
        scratch_types=[
            pltpu.VMEM((_CHUNK,), jnp.int32),
            pltpu.VMEM((_CHUNK, _D), jnp.float32),
            pltpu.VMEM((_CHUNK, _D), jnp.float32),
            pltpu.VMEM_SHARED((_NPAD, _D), jnp.float32),
        ],
        **_SC_PARAMS,
    )
    def k(dst_hbm, deg_hbm, idx_v, ones_v, zrows_v, acc_sp):
        c = lax.axis_index("c")
        s = lax.axis_index("s")
        w = c * _NS + s
        _fill_rows(ones_v, _CHUNK, _D, 1.0)
        _fill_rows(zrows_v, _CHUNK, _D, 0.0)
        _zero_acc_slice(zrows_v, acc_sp, s)
        plsc.subcore_barrier()

        def chunk(k0, _):
            base = (w * _CPW + k0) * _CHUNK
            pltpu.sync_copy(dst_hbm.at[pl.ds(base, _CHUNK)], idx_v)
            pltpu.sync_copy(ones_v, acc_sp.at[idx_v], add=True)
            return 0
        lax.fori_loop(0, _CPW, chunk, 0)
        plsc.subcore_barrier()
        pltpu.sync_copy(acc_sp.at[pl.ds(s * _RB, _RB)],
                        deg_hbm.at[pl.ds(c * _NPAD + s * _RB, _RB)])
    return k(dstp)


def _sc_gcn_agg(srcp, dstp, gtab):
    @functools.partial(
        pl.kernel,
        out_type=jax.ShapeDtypeStruct((_NC * _NPAD, _D), jnp.float32),
        mesh=_sc_mesh(),
        scratch_types=[
            pltpu.VMEM((_CHUNK,), jnp.int32),
            pltpu.VMEM((_CHUNK,), jnp.int32),
            pltpu.VMEM((_CHUNK,), jnp.int32),
            pltpu.VMEM((_CHUNK,), jnp.int32),
            pltpu.VMEM((_CHUNK, _D), jnp.float32),
            pltpu.VMEM((_CHUNK, _D), jnp.float32),
            pltpu.VMEM_SHARED((_NPAD, _D), jnp.float32),
            pltpu.SemaphoreType.DMA,
            pltpu.SemaphoreType.DMA,
        ],
        **_SC_PARAMS,
    )
    def k(src_hbm, dst_hbm, gtab_hbm, out_hbm,
          sidx_a, didx_a, sidx_b, didx_b, rows_a, rows_b, acc_sp,
          sem_a, sem_b):
        c = lax.axis_index("c")
        s = lax.axis_index("s")
        w = c * _NS + s
        _fill_rows(rows_a, _CHUNK, _D, 0.0)
        _zero_acc_slice(rows_a, acc_sp, s)
        plsc.subcore_barrier()

        def pair(k2, _):
            base_a = (w * _CPW + 2 * k2) * _CHUNK
            base_b = base_a + _CHUNK
            pltpu.sync_copy(src_hbm.at[pl.ds(base_a, _CHUNK)], sidx_a)
            pltpu.sync_copy(dst_hbm.at[pl.ds(base_a, _CHUNK)], didx_a)
            da = pltpu.async_copy(gtab_hbm.at[sidx_a], rows_a, sem_a)
            pltpu.sync_copy(src_hbm.at[pl.ds(base_b, _CHUNK)], sidx_b)
            pltpu.sync_copy(dst_hbm.at[pl.ds(base_b, _CHUNK)], didx_b)
            db = pltpu.async_copy(gtab_hbm.at[sidx_b], rows_b, sem_b)
            da.wait()
            pltpu.sync_copy(rows_a, acc_sp.at[didx_a], add=True)
            db.wait()
            pltpu.sync_copy(rows_b, acc_sp.at[didx_b], add=True)
            return 0
        lax.fori_loop(0, _CPW // 2, pair, 0)
        plsc.subcore_barrier()
        pltpu.sync_copy(acc_sp.at[pl.ds(s * _RB, _RB)],
                        out_hbm.at[pl.ds(c * _NPAD + s * _RB, _RB)])
    return k(srcp, dstp, gtab)


def _sc_gat_den(srcp, dstp, as_t, ad_t):
    nrow = _NPAD // 2
    @functools.partial(
        pl.kernel,
        out_type=(jax.ShapeDtypeStruct((_NC * nrow, _D), jnp.float32),
                  jax.ShapeDtypeStruct((_HEADS, _EPAD), jnp.float32)),
        mesh=_sc_mesh(),
        scratch_types=[
            pltpu.VMEM((2 * _NPAD,), jnp.float32),
            pltpu.VMEM((2 * _NPAD,), jnp.float32),
            pltpu.VMEM((_CHUNK,), jnp.int32),
            pltpu.VMEM((_CHUNK,), jnp.int32),
            pltpu.VMEM((_CHUNK,), jnp.int32),
            pltpu.VMEM((_CHUNK,), jnp.int32),
            pltpu.VMEM((_CHUNK,), jnp.int32),
            pltpu.VMEM((_CHUNK,), jnp.int32),
            pltpu.VMEM((_CHUNK, _D), jnp.float32),
            pltpu.VMEM((_CHUNK, _D), jnp.float32),
            pltpu.VMEM((2, _CHUNK), jnp.float32),
            pltpu.VMEM((2, _CHUNK), jnp.float32),
            pltpu.VMEM_SHARED((nrow, _D), jnp.float32),
            pltpu.SemaphoreType.DMA,
            pltpu.SemaphoreType.DMA,
        ],
        **_SC_PARAMS,
    )
    def k(src_hbm, dst_hbm, as_hbm, ad_hbm, den_hbm, p_hbm,
          as_v, ad_v, sidx_a, didx_a, didx2_a, sidx_b, didx_b, didx2_b,
          pden_a, pden_b, pst_a, pst_b, den_sp, sem_a, sem_b):
        c = lax.axis_index("c")
        s = lax.axis_index("s")
        rbd = nrow // _NS
        for hh in range(2):
            off = (c * 2 + hh) * _NPAD
            pltpu.sync_copy(as_hbm.at[pl.ds(off, _NPAD)],
                            as_v.at[pl.ds(hh * _NPAD, _NPAD)])
            pltpu.sync_copy(ad_hbm.at[pl.ds(off, _NPAD)],
                            ad_v.at[pl.ds(hh * _NPAD, _NPAD)])
        _fill_rows(pden_a, _CHUNK, _D, 0.0)
        _fill_rows(pden_b, _CHUNK, _D, 0.0)
        for r in range(rbd // 64):
            pltpu.sync_copy(pden_a.at[pl.ds(0, 64)],
                            den_sp.at[pl.ds(s * rbd + r * 64, 64)])
        plsc.subcore_barrier()
        lanes = lax.iota(jnp.int32, 16)
        zero16 = jnp.zeros((16,), jnp.float32)
        npt = _NCH // _NS

        def do_chunk(base, sidx_v, didx_v, didx2_v, pden_v, pst_v):
            pltpu.sync_copy(src_hbm.at[pl.ds(base, _CHUNK)], sidx_v)
            pltpu.sync_copy(dst_hbm.at[pl.ds(base, _CHUNK)], didx_v)
            for j in range(_CHUNK // 16):
                s16 = sidx_v[pl.ds(j * 16, 16)]
                d16 = didx_v[pl.ds(j * 16, 16)]
                nr16 = jnp.full((16,), nrow, jnp.int32)
                hi = d16 >= nr16
                didx2_v[pl.ds(j * 16, 16)] = jnp.where(hi, d16 - nr16, d16)
                par2 = jnp.where(hi, 2, 0)
                rows16 = lanes + (j * 16)
                for hh in range(2):
                    hoff = jnp.full((16,), hh * _NPAD, jnp.int32)
                    e = (plsc.load_gather(as_v, [s16 + hoff])
                         + plsc.load_gather(ad_v, [d16 + hoff]))
                    e = jnp.where(e >= 0, e, 0.2 * e)
                    p16 = jnp.exp(e)
                    pst_v[hh, pl.ds(j * 16, 16)] = p16
                    hv = jnp.full((16,), hh, jnp.int32)
                    plsc.store_scatter(pden_v, [rows16, par2 + hv], p16)
                    plsc.store_scatter(pden_v, [rows16, (2 - par2) + hv],
                                       zero16)
            for hh in range(2):
                pltpu.sync_copy(
                    pst_v.at[hh],
                    p_hbm.at[c * 2 + hh].at[pl.ds(base, _CHUNK)])

        def pair(k2, _):
            base_a = (s * npt + 2 * k2) * _CHUNK
            base_b = base_a + _CHUNK
            do_chunk(base_a, sidx_a, didx_a, didx2_a, pden_a, pst_a)
            da = pltpu.async_copy(pden_a, den_sp.at[didx2_a], sem_a,
                                  add=True)
            do_chunk(base_b, sidx_b, didx_b, didx2_b, pden_b, pst_b)
            db = pltpu.async_copy(pden_b, den_sp.at[didx2_b], sem_b,
                                  add=True)
            da.wait()
            db.wait()
            return 0
        lax.fori_loop(0, npt // 2, pair, 0)
        plsc.subcore_barrier()
        pltpu.sync_copy(den_sp.at[pl.ds(s * rbd, rbd)],
                        den_hbm.at[pl.ds(c * nrow + s * rbd, rbd)])
    return k(srcp, dstp, as_t, ad_t)


def _sc_gat_agg(srcp2d, dstp2d, p2d, htab):
    nrows = _EPAD // _ACH
    rpt = nrows // _NC // _NS
    hreg = 32
    @functools.partial(
        pl.kernel,
        out_type=jax.ShapeDtypeStruct((_HEADS, _NC * _NPAD, _HID),
                                      jnp.float32),
        mesh=_sc_mesh(),
        scratch_types=[
            pltpu.VMEM((hreg, _ACH), jnp.int32),
            pltpu.VMEM((hreg, _ACH), jnp.int32),
            pltpu.VMEM((hreg, _ACH), jnp.float32),
            pltpu.VMEM((_ACH, _HID), jnp.float32),
            pltpu.VMEM((_ACH, _HID), jnp.float32),
            pltpu.VMEM((_ACH, _HID), jnp.float32),
            pltpu.VMEM_SHARED((_NPAD, _HID), jnp.float32),
            [pltpu.SemaphoreType.DMA] * 4,
            [pltpu.SemaphoreType.DMA] * 4,
        ],
        **_SC_PARAMS,
    )
    def k(src_hbm, dst_hbm, p_hbm, htab_hbm, out_hbm,
          src_v, dst_v, p_v, rows_0, rows_1, rows_2, acc_sp,
          gsems, ssems):
        c = lax.axis_index("c")
        s = lax.axis_index("s")
        rstart = c * (nrows // _NC) + s * rpt
        rows_bufs = [rows_0, rows_1, rows_2, rows_0]

        def scale_rows(j, rows_v):
            def mul16(j16):
                p16 = p_v[j, pl.ds(j16 * 16, 16)]
                for i in range(16):
                    iv = jnp.full((16,), i, jnp.int32)
                    pv = p16.at[iv].get(mode="promise_in_bounds")
                    r = j16 * 16 + i
                    for q in range(_HID // 16):
                        sl = pl.ds(q * 16, 16)
                        rows_v[r, sl] = rows_v[r, sl] * pv
            plsc.parallel_loop(0, _ACH // 16)(mul16)

        def do_head(h, _):
            _fill_rows(rows_0, _ACH, _HID, 0.0)
            for r in range(_RB // _ACH):
                pltpu.sync_copy(
                    rows_0, acc_sp.at[pl.ds(s * _RB + r * _ACH, _ACH)])
            plsc.subcore_barrier()

            def do_reg(reg, _):
                rs = rstart + reg * hreg
                pltpu.sync_copy(src_hbm.at[pl.ds(rs, hreg)], src_v)
                pltpu.sync_copy(dst_hbm.at[pl.ds(rs, hreg)], dst_v)
                pltpu.sync_copy(p_hbm.at[h].at[pl.ds(rs, hreg)], p_v)

                def quad(k4, _):
                    js = [4 * k4 + q for q in range(4)]
                    gs = [pltpu.async_copy(
                        htab_hbm.at[h].at[src_v.at[js[q]]],
                        rows_bufs[q], gsems[q]) for q in range(3)]
                    ss = []
                    for q in range(3):
                        gs[q].wait()
                        scale_rows(js[q], rows_bufs[q])
                        ss.append(pltpu.async_copy(
                            rows_bufs[q], acc_sp.at[dst_v.at[js[q]]],
                            ssems[q], add=True))
                        if q == 0:
                            ss[0].wait()
                            gs.append(pltpu.async_copy(
                                htab_hbm.at[h].at[src_v.at[js[3]]],
                                rows_0, gsems[3]))
                    gs[3].wait()
                    scale_rows(js[3], rows_0)
                    ss.append(pltpu.async_copy(
                        rows_0, acc_sp.at[dst_v.at[js[3]]],
                        ssems[3], add=True))
                    for q in range(1, 4):
                        ss[q].wait()
                    return 0
                lax.fori_loop(0, hreg // 4, quad, 0)
                return 0
            lax.fori_loop(0, rpt // hreg, do_reg, 0)
            plsc.subcore_barrier()
            pltpu.sync_copy(
                acc_sp.at[pl.ds(s * _RB, _RB)],
                out_hbm.at[h].at[pl.ds(c * _NPAD + s * _RB, _RB)])
            plsc.subcore_barrier()
            return 0
        lax.fori_loop(0, _HEADS, do_head, 0)
    return k(srcp2d, dstp2d, p2d, htab)


_HIGH = jax.lax.Precision.HIGHEST


def _dot(a, b):
    return jnp.dot(a, b, preferred_element_type=jnp.float32,
                   precision=_HIGH)


def _dinv_of(deg_ref, i):
    deg = (deg_ref[0, pl.ds(i * 640, 640)][:, :1]
           + deg_ref[1, pl.ds(i * 640, 640)][:, :1] + 1.0)
    return lax.rsqrt(deg)


def _tc1(deg2, xp, W1):
    def body(deg_ref, x_ref, w_ref, h1_ref, g1_ref):
        i = pl.program_id(0)
        dinv = _dinv_of(deg_ref, i)
        h = _dot(x_ref[...], w_ref[...])
        h1_ref[...] = h
        g1_ref[...] = h * dinv
    return pl.pallas_call(
        body,
        grid=(_GRID,),
        in_specs=[
            pl.BlockSpec((_NC, _NPAD, _D), lambda i: (0, 0, 0)),
            pl.BlockSpec((640, _D), lambda i: (i, 0)),
            pl.BlockSpec((_D, _D), lambda i: (0, 0)),
        ],
        out_specs=[
            pl.BlockSpec((640, _D), lambda i: (i, 0)),
            pl.BlockSpec((640, _D), lambda i: (i, 0)),
        ],
        out_shape=[
            jax.ShapeDtypeStruct((_NPAD, _D), jnp.float32),
            jax.ShapeDtypeStruct((_NPAD, _D), jnp.float32),
        ],
    )(deg2, xp, W1)


def _tc2(deg2, S1, h1, b1, Wa1, asrc1, adst1):
    def body(deg_ref, s1_ref, h1_ref, b_ref, w_ref, as_ref, ad_ref,
             h2_ref, as2_ref, ad2_ref):
        i = pl.program_id(0)
        dinv = _dinv_of(deg_ref, i)
        ssum = s1_ref[0] + s1_ref[1]
        h1v = h1_ref[...]
        x2 = jnp.maximum(
            dinv * ssum + (dinv * dinv) * h1v + b_ref[...], 0.0)
        arows, drows = [], []
        for h in range(_HEADS):
            h2 = _dot(x2, w_ref[:, h * _HID:(h + 1) * _HID])
            h2_ref[h] = h2
            arows.append(jnp.sum(h2 * as_ref[h][None, :], axis=1)[None, :])
            drows.append(jnp.sum(h2 * ad_ref[h][None, :], axis=1)[None, :])
        as2_ref[...] = jnp.concatenate(arows, axis=0)
        ad2_ref[...] = jnp.concatenate(drows, axis=0)
    return pl.pallas_call(
        body,
        grid=(_GRID,),
        in_specs=[
            pl.BlockSpec((_NC, _NPAD, _D), lambda i: (0, 0, 0)),
            pl.BlockSpec((_NC, 640, _D), lambda i: (0, i, 0)),
            pl.BlockSpec((640, _D), lambda i: (i, 0)),
            pl.BlockSpec((1, _D), lambda i: (0, 0)),
            pl.BlockSpec((_D, _HEADS * _HID), lambda i: (0, 0)),
            pl.BlockSpec((_HEADS, _HID), lambda i: (0, 0)),
            pl.BlockSpec((_HEADS, _HID), lambda i: (0, 0)),
        ],
        out_specs=[
            pl.BlockSpec((_HEADS, 640, _HID), lambda i: (0, i, 0)),
            pl.BlockSpec((_HEADS, 640), lambda i: (0, i)),
            pl.BlockSpec((_HEADS, 640), lambda i: (0, i)),
        ],
        out_shape=[
            jax.ShapeDtypeStruct((_HEADS, _NPAD, _HID), jnp.float32),
            jax.ShapeDtypeStruct((_HEADS, _NPAD), jnp.float32),
            jax.ShapeDtypeStruct((_HEADS, _NPAD), jnp.float32),
        ],
    )(deg2, S1, h1, b1, Wa1, asrc1, adst1)


def _gat_epilogue(S_ref, den_ref, hh_ref, asv_ref, adv_ref, bias_ref):
    in_lo = pl.program_id(0) < (_NPAD // 2 // 640)
    parts = []
    for h in range(_HEADS):
        es = asv_ref[h, :] + adv_ref[h, :]
        ps = jnp.exp(jnp.where(es >= 0, es, 0.2 * es))[:, None]
        hv = hh_ref[h]
        a = den_ref[h // 2]
        lo = a[:, (h % 2):(h % 2) + 1]
        hi = a[:, 2 + (h % 2):3 + (h % 2)]
        den = jnp.where(in_lo, lo, hi) + ps
        num = S_ref[h, 0] + S_ref[h, 1] + ps * hv
        parts.append(jnp.maximum(
            num / den + bias_ref[0, h * _HID:(h + 1) * _HID][None, :], 0.0))
    return jnp.concatenate(parts, axis=1)


def _tc3(S2, den1, h2h, as2, ad2, ba1, Wa2, asrc2, adst2):
    def body(s2_ref, den_ref, h2_ref, as2_ref, ad2_ref, b_ref, w_ref,
             asw_ref, adw_ref, h3_ref, as3_ref, ad3_ref):
        x3 = _gat_epilogue(s2_ref, den_ref, h2_ref, as2_ref, ad2_ref,
                           b_ref)
        arows, drows = [], []
        for g in range(_HEADS):
            h3 = _dot(x3, w_ref[:, g * _HID:(g + 1) * _HID])
            h3_ref[g] = h3
            arows.append(jnp.sum(h3 * asw_ref[g][None, :], axis=1)[None, :])
            drows.append(jnp.sum(h3 * adw_ref[g][None, :], axis=1)[None, :])
        as3_ref[...] = jnp.concatenate(arows, axis=0)
        ad3_ref[...] = jnp.concatenate(drows, axis=0)
    return pl.pallas_call(
        body,
        grid=(_GRID,),
        in_specs=[
            pl.BlockSpec((_HEADS, _NC, 640, _HID), lambda i: (0, 0, i, 0)),
            pl.BlockSpec((_NC, 640, _D), lambda i: (0, i % 8, 0)),
            pl.BlockSpec((_HEADS, 640, _HID), lambda i: (0, i, 0)),
            pl.BlockSpec((_HEADS, 640), lambda i: (0, i)),
            pl.BlockSpec((_HEADS, 640), lambda i: (0, i)),
            pl.BlockSpec((1, _HEADS * _HID), lambda i: (0, 0)),
            pl.BlockSpec((_HEADS * _HID, _HEADS * _HID), lambda i: (0, 0)),
            pl.BlockSpec((_HEADS, _HID), lambda i: (0, 0)),
            pl.BlockSpec((_HEADS, _HID), lambda i: (0, 0)),
        ],
        out_specs=[
            pl.BlockSpec((_HEADS, 640, _HID), lambda i: (0, i, 0)),
            pl.BlockSpec((_HEADS, 640), lambda i: (0, i)),
            pl.BlockSpec((_HEADS, 640), lambda i: (0, i)),
        ],
        out_shape=[
            jax.ShapeDtypeStruct((_HEADS, _NPAD, _HID), jnp.float32),
            jax.ShapeDtypeStruct((_HEADS, _NPAD), jnp.float32),
            jax.ShapeDtypeStruct((_HEADS, _NPAD), jnp.float32),
        ],
    )(S2, den1, h2h, as2, ad2, ba1, Wa2, asrc2, adst2)


def _tc4(deg2, S3, den2, h3h, as3, ad3, ba2, W2):
    def body(deg_ref, s3_ref, den_ref, h3_ref, as3_ref, ad3_ref, b_ref,
             w_ref, h4_ref, g4_ref):
        i = pl.program_id(0)
        x4 = _gat_epilogue(s3_ref, den_ref, h3_ref, as3_ref, ad3_ref,
                           b_ref)
        h4 = _dot(x4, w_ref[...])
        h4_ref[...] = h4
        dinv = _dinv_of(deg_ref, i)
        g4_ref[...] = h4 * dinv
    return pl.pallas_call(
        body,
        grid=(_GRID,),
        in_specs=[
            pl.BlockSpec((_NC, _NPAD, _D), lambda i: (0, 0, 0)),
            pl.BlockSpec((_HEADS, _NC, 640, _HID), lambda i: (0, 0, i, 0)),
            pl.BlockSpec((_NC, 640, _D), lambda i: (0, i % 8, 0)),
            pl.BlockSpec((_HEADS, 640, _HID), lambda i: (0, i, 0)),
            pl.BlockSpec((_HEADS, 640), lambda i: (0, i)),
            pl.BlockSpec((_HEADS, 640), lambda i: (0, i)),
            pl.BlockSpec((1, _HEADS * _HID), lambda i: (0, 0)),
            pl.BlockSpec((_HEADS * _HID, _D), lambda i: (0, 0)),
        ],
        out_specs=[
            pl.BlockSpec((640, _D), lambda i: (i, 0)),
            pl.BlockSpec((640, _D), lambda i: (i, 0)),
        ],
        out_shape=[
            jax.ShapeDtypeStruct((_NPAD, _D), jnp.float32),
            jax.ShapeDtypeStruct((_NPAD, _D), jnp.float32),
        ],
    )(deg2, S3, den2, h3h, as3, ad3, ba2, W2)


def _tc5(deg2, S4, h4, b2):
    def body(deg_ref, s4_ref, h4_ref, b_ref, out_ref):
        i = pl.program_id(0)
        dinv = _dinv_of(deg_ref, i)
        ssum = s4_ref[0] + s4_ref[1]
        out_ref[...] = (dinv * ssum + (dinv * dinv) * h4_ref[...]
                        + b_ref[...])
    return pl.pallas_call(
        body,
        grid=(_GRID,),
        in_specs=[
            pl.BlockSpec((_NC, _NPAD, _D), lambda i: (0, 0, 0)),
            pl.BlockSpec((_NC, 640, _D), lambda i: (0, i, 0)),
            pl.BlockSpec((640, _D), lambda i: (i, 0)),
            pl.BlockSpec((1, _D), lambda i: (0, 0)),
        ],
        out_specs=pl.BlockSpec((640, _D), lambda i: (i, 0)),
        out_shape=jax.ShapeDtypeStruct((_NPAD, _D), jnp.float32),
    )(deg2, S4, h4, b2)


def kernel(x, edge_index, batch, W1, b1, Wa1, asrc1, adst1, ba1,
           Wa2, asrc2, adst2, ba2, W2, b2):
    del batch
    f32 = jnp.float32
    xp = jnp.pad(x, ((0, _NPAD - _N), (0, 0)))
    epad = _N + (jnp.arange(_EPAD - _E, dtype=jnp.int32) % (_NPAD - _N))
    srcp = jnp.concatenate([edge_index[0].astype(jnp.int32), epad])
    dstp = jnp.concatenate([edge_index[1].astype(jnp.int32), epad])
    b1r = b1.reshape(1, _D).astype(f32)
    ba1r = ba1.reshape(1, _HEADS * _HID).astype(f32)
    ba2r = ba2.reshape(1, _HEADS * _HID).astype(f32)
    b2r = b2.reshape(1, _D).astype(f32)

    deg2 = _sc_hist(dstp).reshape(_NC, _NPAD, _D)

    h1, g1 = _tc1(deg2, xp, W1)
    S1 = _sc_gcn_agg(srcp, dstp, g1).reshape(_NC, _NPAD, _D)

    h2h, as2, ad2 = _tc2(deg2, S1, h1, b1r, Wa1, asrc1, adst1)
    srcp2d = srcp.reshape(_EPAD // _ACH, _ACH)
    dstp2d = dstp.reshape(_EPAD // _ACH, _ACH)
    as2f, ad2f = as2.reshape(-1), ad2.reshape(-1)
    den1, p1 = _sc_gat_den(srcp, dstp, as2f, ad2f)
    den1 = den1.reshape(_NC, _NPAD // 2, _D)
    S2 = _sc_gat_agg(srcp2d, dstp2d,
                     p1.reshape(_HEADS, _EPAD // _ACH, _ACH), h2h)
    S2 = S2.reshape(_HEADS, _NC, _NPAD, _HID)
    h3h, as3, ad3 = _tc3(S2, den1, h2h, as2, ad2, ba1r, Wa2, asrc2, adst2)

    as3f, ad3f = as3.reshape(-1), ad3.reshape(-1)
    den2, p2 = _sc_gat_den(srcp, dstp, as3f, ad3f)
    den2 = den2.reshape(_NC, _NPAD // 2, _D)
    S3 = _sc_gat_agg(srcp2d, dstp2d,
                     p2.reshape(_HEADS, _EPAD // _ACH, _ACH), h3h)
    S3 = S3.reshape(_HEADS, _NC, _NPAD, _HID)

    h4, g4 = _tc4(deg2, S3, den2, h3h, as3, ad3, ba2r, W2)
    S4 = _sc_gcn_agg(srcp, dstp, g4).reshape(_NC, _NPAD, _D)
    out = _tc5(deg2, S4, h4, b2r)
    return out[:_N]


# speedup vs baseline: 1.0550x; 1.0550x over previous
import functools

import jax
import jax.numpy as jnp
from jax import lax
from jax.experimental import pallas as pl
from jax.experimental.pallas import tpu as pltpu
from jax.experimental.pallas import tpu_sc as plsc

_HEADS = 4
_HID = 128
_D = 128
_N = 10000
_E = 320000
_NPAD = 10240
_EPAD = 327680
_CHUNK = 128
_NCH = _EPAD // _CHUNK
_NC = 2
_NS = 16
_NW = _NC * _NS
_CPW = _NCH // _NW
_CPT = _NCH // _NC // _NS
_RB = _NPAD // _NS
_DCH = 32
_ACH = 80
_GRID = _NPAD // 640


def _sc_mesh():
    return plsc.VectorSubcoreMesh(
        core_axis_name="c", subcore_axis_name="s",
        num_cores=_NC, num_subcores=_NS)

_SC_PARAMS = dict(
    compiler_params=pltpu.CompilerParams(needs_layout_passes=False))


def _fill_rows(ref, rows, width, value):
    def body(i, _):
        for q in range(width // 16):
            ref[i, pl.ds(q * 16, 16)] = jnp.full((16,), value, jnp.float32)
        return 0
    lax.fori_loop(0, rows, body, 0)


def _zero_acc_slice(zrows_v, acc_sp, s):
    for r in range(_RB // _CHUNK):
        pltpu.sync_copy(zrows_v,
                        acc_sp.at[pl.ds(s * _RB + r * _CHUNK, _CHUNK)])


def _sc_hist(dstp):
    @functools.partial(
        pl.kernel,
        out_type=jax.ShapeDtypeStruct((_NC * _NPAD, _D), jnp.float32),
        mesh=_sc_mesh(),
        scratch_types=[
            pltpu.VMEM((_CHUNK,), jnp.int32),
            pltpu.VMEM((_CHUNK, _D), jnp.float32),
            pltpu.VMEM((_CHUNK, _D), jnp.float32),
            pltpu.VMEM_SHARED((_NPAD, _D), jnp.float32),
        ],
        **_SC_PARAMS,
    )
    def k(dst_hbm, deg_hbm, idx_v, ones_v, zrows_v, acc_sp):
        c = lax.axis_index("c")
        s = lax.axis_index("s")
        w = c * _NS + s
        _fill_rows(ones_v, _CHUNK, _D, 1.0)
        _fill_rows(zrows_v, _CHUNK, _D, 0.0)
        _zero_acc_slice(zrows_v, acc_sp, s)
        plsc.subcore_barrier()

        def chunk(k0, _):
            base = (w * _CPW + k0) * _CHUNK
            pltpu.sync_copy(dst_hbm.at[pl.ds(base, _CHUNK)], idx_v)
            pltpu.sync_copy(ones_v, acc_sp.at[idx_v], add=True)
            return 0
        lax.fori_loop(0, _CPW, chunk, 0)
        plsc.subcore_barrier()
        pltpu.sync_copy(acc_sp.at[pl.ds(s * _RB, _RB)],
                        deg_hbm.at[pl.ds(c * _NPAD + s * _RB, _RB)])
    return k(dstp)


def _sc_gcn_agg(srcp, dstp, gtab):
    @functools.partial(
        pl.kernel,
        out_type=jax.ShapeDtypeStruct((_NC * _NPAD, _D), jnp.float32),
        mesh=_sc_mesh(),
        scratch_types=[
            pltpu.VMEM((_CHUNK,), jnp.int32),
            pltpu.VMEM((_CHUNK,), jnp.int32),
            pltpu.VMEM((_CHUNK,), jnp.int32),
            pltpu.VMEM((_CHUNK,), jnp.int32),
            pltpu.VMEM((_CHUNK, _D), jnp.float32),
            pltpu.VMEM((_CHUNK, _D), jnp.float32),
            pltpu.VMEM_SHARED((_NPAD, _D), jnp.float32),
            pltpu.SemaphoreType.DMA,
            pltpu.SemaphoreType.DMA,
        ],
        **_SC_PARAMS,
    )
    def k(src_hbm, dst_hbm, gtab_hbm, out_hbm,
          sidx_a, didx_a, sidx_b, didx_b, rows_a, rows_b, acc_sp,
          sem_a, sem_b):
        c = lax.axis_index("c")
        s = lax.axis_index("s")
        w = c * _NS + s
        _fill_rows(rows_a, _CHUNK, _D, 0.0)
        _zero_acc_slice(rows_a, acc_sp, s)
        plsc.subcore_barrier()

        def pair(k2, _):
            base_a = (w * _CPW + 2 * k2) * _CHUNK
            base_b = base_a + _CHUNK
            pltpu.sync_copy(src_hbm.at[pl.ds(base_a, _CHUNK)], sidx_a)
            pltpu.sync_copy(dst_hbm.at[pl.ds(base_a, _CHUNK)], didx_a)
            da = pltpu.async_copy(gtab_hbm.at[sidx_a], rows_a, sem_a)
            pltpu.sync_copy(src_hbm.at[pl.ds(base_b, _CHUNK)], sidx_b)
            pltpu.sync_copy(dst_hbm.at[pl.ds(base_b, _CHUNK)], didx_b)
            db = pltpu.async_copy(gtab_hbm.at[sidx_b], rows_b, sem_b)
            da.wait()
            pltpu.sync_copy(rows_a, acc_sp.at[didx_a], add=True)
            db.wait()
            pltpu.sync_copy(rows_b, acc_sp.at[didx_b], add=True)
            return 0
        lax.fori_loop(0, _CPW // 2, pair, 0)
        plsc.subcore_barrier()
        pltpu.sync_copy(acc_sp.at[pl.ds(s * _RB, _RB)],
                        out_hbm.at[pl.ds(c * _NPAD + s * _RB, _RB)])
    return k(srcp, dstp, gtab)


def _sc_gat_den(srcp, dstp, as_t, ad_t):
    nrow = _NPAD // 2
    @functools.partial(
        pl.kernel,
        out_type=(jax.ShapeDtypeStruct((_NC * nrow, _D), jnp.float32),
                  jax.ShapeDtypeStruct((_HEADS, _EPAD), jnp.float32)),
        mesh=_sc_mesh(),
        scratch_types=[
            pltpu.VMEM((2 * _NPAD,), jnp.float32),
            pltpu.VMEM((2 * _NPAD,), jnp.float32),
            pltpu.VMEM((_CHUNK,), jnp.int32),
            pltpu.VMEM((_CHUNK,), jnp.int32),
            pltpu.VMEM((_CHUNK,), jnp.int32),
            pltpu.VMEM((_CHUNK,), jnp.int32),
            pltpu.VMEM((_CHUNK,), jnp.int32),
            pltpu.VMEM((_CHUNK,), jnp.int32),
            pltpu.VMEM((_CHUNK, _D), jnp.float32),
            pltpu.VMEM((_CHUNK, _D), jnp.float32),
            pltpu.VMEM((2, _CHUNK), jnp.float32),
            pltpu.VMEM((2, _CHUNK), jnp.float32),
            pltpu.VMEM_SHARED((nrow, _D), jnp.float32),
            pltpu.SemaphoreType.DMA,
            pltpu.SemaphoreType.DMA,
        ],
        **_SC_PARAMS,
    )
    def k(src_hbm, dst_hbm, as_hbm, ad_hbm, den_hbm, p_hbm,
          as_v, ad_v, sidx_a, didx_a, didx2_a, sidx_b, didx_b, didx2_b,
          pden_a, pden_b, pst_a, pst_b, den_sp, sem_a, sem_b):
        c = lax.axis_index("c")
        s = lax.axis_index("s")
        rbd = nrow // _NS
        for hh in range(2):
            off = (c * 2 + hh) * _NPAD
            pltpu.sync_copy(as_hbm.at[pl.ds(off, _NPAD)],
                            as_v.at[pl.ds(hh * _NPAD, _NPAD)])
            pltpu.sync_copy(ad_hbm.at[pl.ds(off, _NPAD)],
                            ad_v.at[pl.ds(hh * _NPAD, _NPAD)])
        _fill_rows(pden_a, _CHUNK, _D, 0.0)
        _fill_rows(pden_b, _CHUNK, _D, 0.0)
        for r in range(rbd // 64):
            pltpu.sync_copy(pden_a.at[pl.ds(0, 64)],
                            den_sp.at[pl.ds(s * rbd + r * 64, 64)])
        plsc.subcore_barrier()
        lanes = lax.iota(jnp.int32, 16)
        zero16 = jnp.zeros((16,), jnp.float32)
        npt = _NCH // _NS

        def do_chunk(base, sidx_v, didx_v, didx2_v, pden_v, pst_v):
            pltpu.sync_copy(src_hbm.at[pl.ds(base, _CHUNK)], sidx_v)
            pltpu.sync_copy(dst_hbm.at[pl.ds(base, _CHUNK)], didx_v)
            for j in range(_CHUNK // 16):
                s16 = sidx_v[pl.ds(j * 16, 16)]
                d16 = didx_v[pl.ds(j * 16, 16)]
                nr16 = jnp.full((16,), nrow, jnp.int32)
                hi = d16 >= nr16
                didx2_v[pl.ds(j * 16, 16)] = jnp.where(hi, d16 - nr16, d16)
                par2 = jnp.where(hi, 2, 0)
                rows16 = lanes + (j * 16)
                for hh in range(2):
                    hoff = jnp.full((16,), hh * _NPAD, jnp.int32)
                    e = (plsc.load_gather(as_v, [s16 + hoff])
                         + plsc.load_gather(ad_v, [d16 + hoff]))
                    e = jnp.where(e >= 0, e, 0.2 * e)
                    p16 = jnp.exp(e)
                    pst_v[hh, pl.ds(j * 16, 16)] = p16
                    hv = jnp.full((16,), hh, jnp.int32)
                    plsc.store_scatter(pden_v, [rows16, par2 + hv], p16)
                    plsc.store_scatter(pden_v, [rows16, (2 - par2) + hv],
                                       zero16)
            for hh in range(2):
                pltpu.sync_copy(
                    pst_v.at[hh],
                    p_hbm.at[c * 2 + hh].at[pl.ds(base, _CHUNK)])

        def pair(k2, _):
            base_a = (s * npt + 2 * k2) * _CHUNK
            base_b = base_a + _CHUNK
            do_chunk(base_a, sidx_a, didx_a, didx2_a, pden_a, pst_a)
            da = pltpu.async_copy(pden_a, den_sp.at[didx2_a], sem_a,
                                  add=True)
            do_chunk(base_b, sidx_b, didx_b, didx2_b, pden_b, pst_b)
            db = pltpu.async_copy(pden_b, den_sp.at[didx2_b], sem_b,
                                  add=True)
            da.wait()
            db.wait()
            return 0
        lax.fori_loop(0, npt // 2, pair, 0)
        plsc.subcore_barrier()
        pltpu.sync_copy(den_sp.at[pl.ds(s * rbd, rbd)],
                        den_hbm.at[pl.ds(c * nrow + s * rbd, rbd)])
    return k(srcp, dstp, as_t, ad_t)


def _sc_gat_agg(srcp2d, dstp2d, p2d, htab):
    nrows = _EPAD // _ACH
    rpt = nrows // _NC // _NS
    hreg = rpt // 2
    @functools.partial(
        pl.kernel,
        out_type=jax.ShapeDtypeStruct((_HEADS, _NC * _NPAD, _HID),
                                      jnp.float32),
        mesh=_sc_mesh(),
        scratch_types=[
            pltpu.VMEM((hreg, _ACH), jnp.int32),
            pltpu.VMEM((hreg, _ACH), jnp.int32),
            pltpu.VMEM((hreg, _ACH), jnp.float32),
            pltpu.VMEM((_ACH, _HID), jnp.float32),
            pltpu.VMEM((_ACH, _HID), jnp.float32),
            pltpu.VMEM_SHARED((_NPAD, _HID), jnp.float32),
            pltpu.SemaphoreType.DMA,
            pltpu.SemaphoreType.DMA,
            pltpu.SemaphoreType.DMA,
            pltpu.SemaphoreType.DMA,
        ],
        **_SC_PARAMS,
    )
    def k(src_hbm, dst_hbm, p_hbm, htab_hbm, out_hbm,
          src_v, dst_v, p_v, rows_a, rows_b, acc_sp,
          sem_a, sem_b, sem_sa, sem_sb):
        c = lax.axis_index("c")
        s = lax.axis_index("s")
        rstart = c * (nrows // _NC) + s * rpt

        def scale_rows(j, rows_v):
            def mul16(j16):
                p16 = p_v[j, pl.ds(j16 * 16, 16)]
                for i in range(16):
                    iv = jnp.full((16,), i, jnp.int32)
                    pv = p16.at[iv].get(mode="promise_in_bounds")
                    r = j16 * 16 + i
                    for q in range(_HID // 16):
                        sl = pl.ds(q * 16, 16)
                        rows_v[r, sl] = rows_v[r, sl] * pv
            plsc.parallel_loop(0, _ACH // 16)(mul16)

        def do_head(h, _):
            _fill_rows(rows_a, _ACH, _HID, 0.0)
            for r in range(_RB // _ACH):
                pltpu.sync_copy(
                    rows_a, acc_sp.at[pl.ds(s * _RB + r * _ACH, _ACH)])
            plsc.subcore_barrier()

            def do_half(half, _):
                rs = rstart + half * hreg
                pltpu.sync_copy(src_hbm.at[pl.ds(rs, hreg)], src_v)
                pltpu.sync_copy(dst_hbm.at[pl.ds(rs, hreg)], dst_v)
                pltpu.sync_copy(p_hbm.at[h].at[pl.ds(rs, hreg)], p_v)

                def pair(k2, _):
                    ja = 2 * k2
                    jb = ja + 1
                    da = pltpu.async_copy(
                        htab_hbm.at[h].at[src_v.at[ja]], rows_a, sem_a)
                    db = pltpu.async_copy(
                        htab_hbm.at[h].at[src_v.at[jb]], rows_b, sem_b)
                    da.wait()
                    scale_rows(ja, rows_a)
                    sa = pltpu.async_copy(rows_a, acc_sp.at[dst_v.at[ja]],
                                          sem_sa, add=True)
                    db.wait()
                    scale_rows(jb, rows_b)
                    sb = pltpu.async_copy(rows_b, acc_sp.at[dst_v.at[jb]],
                                          sem_sb, add=True)
                    sa.wait()
                    sb.wait()
                    return 0
                lax.fori_loop(0, hreg // 2, pair, 0)
                return 0
            lax.fori_loop(0, 2, do_half, 0)
            plsc.subcore_barrier()
            pltpu.sync_copy(
                acc_sp.at[pl.ds(s * _RB, _RB)],
                out_hbm.at[h].at[pl.ds(c * _NPAD + s * _RB, _RB)])
            plsc.subcore_barrier()
            return 0
        lax.fori_loop(0, _HEADS, do_head, 0)
    return k(srcp2d, dstp2d, p2d, htab)


_HIGH = jax.lax.Precision.HIGHEST


def _dot(a, b):
    return jnp.dot(a, b, preferred_element_type=jnp.float32,
                   precision=_HIGH)


def _dinv_of(deg_ref, i):
    deg = (deg_ref[0, pl.ds(i * 640, 640)][:, :1]
           + deg_ref[1, pl.ds(i * 640, 640)][:, :1] + 1.0)
    return lax.rsqrt(deg)


def _tc1(deg2, xp, W1):
    def body(deg_ref, x_ref, w_ref, h1_ref, g1_ref):
        i = pl.program_id(0)
        dinv = _dinv_of(deg_ref, i)
        h = _dot(x_ref[...], w_ref[...])
        h1_ref[...] = h
        g1_ref[...] = h * dinv
    return pl.pallas_call(
        body,
        grid=(_GRID,),
        in_specs=[
            pl.BlockSpec((_NC, _NPAD, _D), lambda i: (0, 0, 0)),
            pl.BlockSpec((640, _D), lambda i: (i, 0)),
            pl.BlockSpec((_D, _D), lambda i: (0, 0)),
        ],
        out_specs=[
            pl.BlockSpec((640, _D), lambda i: (i, 0)),
            pl.BlockSpec((640, _D), lambda i: (i, 0)),
        ],
        out_shape=[
            jax.ShapeDtypeStruct((_NPAD, _D), jnp.float32),
            jax.ShapeDtypeStruct((_NPAD, _D), jnp.float32),
        ],
    )(deg2, xp, W1)


def _tc2(deg2, S1, h1, b1, Wa1, asrc1, adst1):
    def body(deg_ref, s1_ref, h1_ref, b_ref, w_ref, as_ref, ad_ref,
             h2_ref, as2_ref, ad2_ref):
        i = pl.program_id(0)
        dinv = _dinv_of(deg_ref, i)
        ssum = s1_ref[0] + s1_ref[1]
        h1v = h1_ref[...]
        x2 = jnp.maximum(
            dinv * ssum + (dinv * dinv) * h1v + b_ref[...], 0.0)
        arows, drows = [], []
        for h in range(_HEADS):
            h2 = _dot(x2, w_ref[:, h * _HID:(h + 1) * _HID])
            h2_ref[h] = h2
            arows.append(jnp.sum(h2 * as_ref[h][None, :], axis=1)[None, :])
            drows.append(jnp.sum(h2 * ad_ref[h][None, :], axis=1)[None, :])
        as2_ref[...] = jnp.concatenate(arows, axis=0)
        ad2_ref[...] = jnp.concatenate(drows, axis=0)
    return pl.pallas_call(
        body,
        grid=(_GRID,),
        in_specs=[
            pl.BlockSpec((_NC, _NPAD, _D), lambda i: (0, 0, 0)),
            pl.BlockSpec((_NC, 640, _D), lambda i: (0, i, 0)),
            pl.BlockSpec((640, _D), lambda i: (i, 0)),
            pl.BlockSpec((1, _D), lambda i: (0, 0)),
            pl.BlockSpec((_D, _HEADS * _HID), lambda i: (0, 0)),
            pl.BlockSpec((_HEADS, _HID), lambda i: (0, 0)),
            pl.BlockSpec((_HEADS, _HID), lambda i: (0, 0)),
        ],
        out_specs=[
            pl.BlockSpec((_HEADS, 640, _HID), lambda i: (0, i, 0)),
            pl.BlockSpec((_HEADS, 640), lambda i: (0, i)),
            pl.BlockSpec((_HEADS, 640), lambda i: (0, i)),
        ],
        out_shape=[
            jax.ShapeDtypeStruct((_HEADS, _NPAD, _HID), jnp.float32),
            jax.ShapeDtypeStruct((_HEADS, _NPAD), jnp.float32),
            jax.ShapeDtypeStruct((_HEADS, _NPAD), jnp.float32),
        ],
    )(deg2, S1, h1, b1, Wa1, asrc1, adst1)


def _gat_epilogue(S_ref, den_ref, hh_ref, asv_ref, adv_ref, bias_ref):
    in_lo = pl.program_id(0) < (_NPAD // 2 // 640)
    parts = []
    for h in range(_HEADS):
        es = asv_ref[h, :] + adv_ref[h, :]
        ps = jnp.exp(jnp.where(es >= 0, es, 0.2 * es))[:, None]
        hv = hh_ref[h]
        a = den_ref[h // 2]
        lo = a[:, (h % 2):(h % 2) + 1]
        hi = a[:, 2 + (h % 2):3 + (h % 2)]
        den = jnp.where(in_lo, lo, hi) + ps
        num = S_ref[h, 0] + S_ref[h, 1] + ps * hv
        parts.append(jnp.maximum(
            num / den + bias_ref[0, h * _HID:(h + 1) * _HID][None, :], 0.0))
    return jnp.concatenate(parts, axis=1)


def _tc3(S2, den1, h2h, as2, ad2, ba1, Wa2, asrc2, adst2):
    def body(s2_ref, den_ref, h2_ref, as2_ref, ad2_ref, b_ref, w_ref,
             asw_ref, adw_ref, h3_ref, as3_ref, ad3_ref):
        x3 = _gat_epilogue(s2_ref, den_ref, h2_ref, as2_ref, ad2_ref,
                           b_ref)
        arows, drows = [], []
        for g in range(_HEADS):
            h3 = _dot(x3, w_ref[:, g * _HID:(g + 1) * _HID])
            h3_ref[g] = h3
            arows.append(jnp.sum(h3 * asw_ref[g][None, :], axis=1)[None, :])
            drows.append(jnp.sum(h3 * adw_ref[g][None, :], axis=1)[None, :])
        as3_ref[...] = jnp.concatenate(arows, axis=0)
        ad3_ref[...] = jnp.concatenate(drows, axis=0)
    return pl.pallas_call(
        body,
        grid=(_GRID,),
        in_specs=[
            pl.BlockSpec((_HEADS, _NC, 640, _HID), lambda i: (0, 0, i, 0)),
            pl.BlockSpec((_NC, 640, _D), lambda i: (0, i % 8, 0)),
            pl.BlockSpec((_HEADS, 640, _HID), lambda i: (0, i, 0)),
            pl.BlockSpec((_HEADS, 640), lambda i: (0, i)),
            pl.BlockSpec((_HEADS, 640), lambda i: (0, i)),
            pl.BlockSpec((1, _HEADS * _HID), lambda i: (0, 0)),
            pl.BlockSpec((_HEADS * _HID, _HEADS * _HID), lambda i: (0, 0)),
            pl.BlockSpec((_HEADS, _HID), lambda i: (0, 0)),
            pl.BlockSpec((_HEADS, _HID), lambda i: (0, 0)),
        ],
        out_specs=[
            pl.BlockSpec((_HEADS, 640, _HID), lambda i: (0, i, 0)),
            pl.BlockSpec((_HEADS, 640), lambda i: (0, i)),
            pl.BlockSpec((_HEADS, 640), lambda i: (0, i)),
        ],
        out_shape=[
            jax.ShapeDtypeStruct((_HEADS, _NPAD, _HID), jnp.float32),
            jax.ShapeDtypeStruct((_HEADS, _NPAD), jnp.float32),
            jax.ShapeDtypeStruct((_HEADS, _NPAD), jnp.float32),
        ],
    )(S2, den1, h2h, as2, ad2, ba1, Wa2, asrc2, adst2)


def _tc4(deg2, S3, den2, h3h, as3, ad3, ba2, W2):
    def body(deg_ref, s3_ref, den_ref, h3_ref, as3_ref, ad3_ref, b_ref,
             w_ref, h4_ref, g4_ref):
        i = pl.program_id(0)
        x4 = _gat_epilogue(s3_ref, den_ref, h3_ref, as3_ref, ad3_ref,
                           b_ref)
        h4 = _dot(x4, w_ref[...])
        h4_ref[...] = h4
        dinv = _dinv_of(deg_ref, i)
        g4_ref[...] = h4 * dinv
    return pl.pallas_call(
        body,
        grid=(_GRID,),
        in_specs=[
            pl.BlockSpec((_NC, _NPAD, _D), lambda i: (0, 0, 0)),
            pl.BlockSpec((_HEADS, _NC, 640, _HID), lambda i: (0, 0, i, 0)),
            pl.BlockSpec((_NC, 640, _D), lambda i: (0, i % 8, 0)),
            pl.BlockSpec((_HEADS, 640, _HID), lambda i: (0, i, 0)),
            pl.BlockSpec((_HEADS, 640), lambda i: (0, i)),
            pl.BlockSpec((_HEADS, 640), lambda i: (0, i)),
            pl.BlockSpec((1, _HEADS * _HID), lambda i: (0, 0)),
            pl.BlockSpec((_HEADS * _HID, _D), lambda i: (0, 0)),
        ],
        out_specs=[
            pl.BlockSpec((640, _D), lambda i: (i, 0)),
            pl.BlockSpec((640, _D), lambda i: (i, 0)),
        ],
        out_shape=[
            jax.ShapeDtypeStruct((_NPAD, _D), jnp.float32),
            jax.ShapeDtypeStruct((_NPAD, _D), jnp.float32),
        ],
    )(deg2, S3, den2, h3h, as3, ad3, ba2, W2)


def _tc5(deg2, S4, h4, b2):
    def body(deg_ref, s4_ref, h4_ref, b_ref, out_ref):
        i = pl.program_id(0)
        dinv = _dinv_of(deg_ref, i)
        ssum = s4_ref[0] + s4_ref[1]
        out_ref[...] = (dinv * ssum + (dinv * dinv) * h4_ref[...]
                        + b_ref[...])
    return pl.pallas_call(
        body,
        grid=(_GRID,),
        in_specs=[
            pl.BlockSpec((_NC, _NPAD, _D), lambda i: (0, 0, 0)),
            pl.BlockSpec((_NC, 640, _D), lambda i: (0, i, 0)),
            pl.BlockSpec((640, _D), lambda i: (i, 0)),
            pl.BlockSpec((1, _D), lambda i: (0, 0)),
        ],
        out_specs=pl.BlockSpec((640, _D), lambda i: (i, 0)),
        out_shape=jax.ShapeDtypeStruct((_NPAD, _D), jnp.float32),
    )(deg2, S4, h4, b2)


def kernel(x, edge_index, batch, W1, b1, Wa1, asrc1, adst1, ba1,
           Wa2, asrc2, adst2, ba2, W2, b2):
    del batch
    f32 = jnp.float32
    xp = jnp.pad(x, ((0, _NPAD - _N), (0, 0)))
    epad = _N + (jnp.arange(_EPAD - _E, dtype=jnp.int32) % (_NPAD - _N))
    srcp = jnp.concatenate([edge_index[0].astype(jnp.int32), epad])
    dstp = jnp.concatenate([edge_index[1].astype(jnp.int32), epad])
    b1r = b1.reshape(1, _D).astype(f32)
    ba1r = ba1.reshape(1, _HEADS * _HID).astype(f32)
    ba2r = ba2.reshape(1, _HEADS * _HID).astype(f32)
    b2r = b2.reshape(1, _D).astype(f32)

    deg2 = _sc_hist(dstp).reshape(_NC, _NPAD, _D)

    h1, g1 = _tc1(deg2, xp, W1)
    S1 = _sc_gcn_agg(srcp, dstp, g1).reshape(_NC, _NPAD, _D)

    h2h, as2, ad2 = _tc2(deg2, S1, h1, b1r, Wa1, asrc1, adst1)
    srcp2d = srcp.reshape(_EPAD // _ACH, _ACH)
    dstp2d = dstp.reshape(_EPAD // _ACH, _ACH)
    as2f, ad2f = as2.reshape(-1), ad2.reshape(-1)
    den1, p1 = _sc_gat_den(srcp, dstp, as2f, ad2f)
    den1 = den1.reshape(_NC, _NPAD // 2, _D)
    S2 = _sc_gat_agg(srcp2d, dstp2d,
                     p1.reshape(_HEADS, _EPAD // _ACH, _ACH), h2h)
    S2 = S2.reshape(_HEADS, _NC, _NPAD, _HID)
    h3h, as3, ad3 = _tc3(S2, den1, h2h, as2, ad2, ba1r, Wa2, asrc2, adst2)

    as3f, ad3f = as3.reshape(-1), ad3.reshape(-1)
    den2, p2 = _sc_gat_den(srcp, dstp, as3f, ad3f)
    den2 = den2.reshape(_NC, _NPAD // 2, _D)
    S3 = _sc_gat_agg(srcp2d, dstp2d,
                     p2.reshape(_HEADS, _EPAD // _ACH, _ACH), h3h)
    S3 = S3.reshape(_HEADS, _NC, _NPAD, _HID)

    h4, g4 = _tc4(deg2, S3, den2, h3h, as3, ad3, ba2r, W2)
    S4 = _sc_gcn_agg(srcp, dstp, g4).reshape(_NC, _NPAD, _D)
    out = _tc5(deg2, S4, h4, b2r)
    return out[:_N]
